# Initial kernel scaffold; baseline (speedup 1.0000x reference)
import jax
import jax.numpy as jnp
from jax.experimental import pallas as pl


def kernel(h, x, edge_index, We1, be1, We2, be2, Winf, binf, Wn1, bn1, Wn2, bn2, ln_g, ln_b):
    raise NotImplementedError("write your pallas kernel here")


# speedup vs baseline: 3.6948x; 3.6948x over previous
import functools

import jax
import jax.numpy as jnp
from jax import lax
from jax.experimental import pallas as pl
from jax.experimental.pallas import tpu as pltpu
from jax.experimental.pallas import tpu_sc as plsc

N = 10000
H = 128
E = 320000
NPAD = 10240
NC = 2
NS = 16
NW = NC * NS
EPT = 10240
E2 = NW * EPT

IW = 128
IRPT = EPT // IW
CG = 256
GG = EPT // CG
CS = 256
GS = EPT // CS
RPT = NPAD // NS

_MESH = plsc.VectorSubcoreMesh(
    core_axis_name="c", subcore_axis_name="s", num_cores=NC, num_subcores=NS)


def _pre_ab_body(h_ref, wt_ref, wb_ref, be1_ref, a_ref, b_ref):
    hb = h_ref[...]
    a_ref[...] = jnp.dot(hb, wt_ref[...], preferred_element_type=jnp.float32) + be1_ref[...]
    b_ref[...] = jnp.dot(hb, wb_ref[...], preferred_element_type=jnp.float32)


def _pre_ab(hp, wt, wb, be1):
    bn = 512
    return pl.pallas_call(
        _pre_ab_body,
        grid=(NPAD // bn,),
        in_specs=[
            pl.BlockSpec((bn, H), lambda i: (i, 0)),
            pl.BlockSpec((H, H), lambda i: (0, 0)),
            pl.BlockSpec((H, H), lambda i: (0, 0)),
            pl.BlockSpec((1, H), lambda i: (0, 0)),
        ],
        out_specs=[
            pl.BlockSpec((bn, H), lambda i: (i, 0)),
            pl.BlockSpec((bn, H), lambda i: (i, 0)),
        ],
        out_shape=[
            jax.ShapeDtypeStruct((NPAD, H), jnp.float32),
            jax.ShapeDtypeStruct((NPAD, H), jnp.float32),
        ],
    )(hp, wt, wb, be1)


@functools.partial(
    pl.kernel,
    out_type=jax.ShapeDtypeStruct((E2,), jnp.float32),
    mesh=_MESH,
    scratch_types=[
        pltpu.VMEM((NPAD,), jnp.float32),
        pltpu.VMEM((NPAD,), jnp.float32),
        pltpu.VMEM((NPAD,), jnp.float32),
        pltpu.VMEM((EPT,), jnp.int32),
        pltpu.VMEM((EPT,), jnp.int32),
        pltpu.VMEM((EPT,), jnp.float32),
    ],
    compiler_params=pltpu.CompilerParams(needs_layout_passes=False),
)
def _dsq_kernel(x0_hbm, x1_hbm, x2_hbm, dst_hbm, src_hbm, dsq_hbm,
                x0v, x1v, x2v, dstv, srcv, dsqv):
    wid = lax.axis_index("s") * NC + lax.axis_index("c")
    base = wid * EPT
    pltpu.sync_copy(x0_hbm, x0v)
    pltpu.sync_copy(x1_hbm, x1v)
    pltpu.sync_copy(x2_hbm, x2v)
    pltpu.sync_copy(dst_hbm.at[pl.ds(base, EPT)], dstv)
    pltpu.sync_copy(src_hbm.at[pl.ds(base, EPT)], srcv)

    def body(t, carry):
        o = t * 16
        di = dstv[pl.ds(o, 16)]
        si = srcv[pl.ds(o, 16)]
        d0 = plsc.load_gather(x0v, [di]) - plsc.load_gather(x0v, [si])
        d1 = plsc.load_gather(x1v, [di]) - plsc.load_gather(x1v, [si])
        d2 = plsc.load_gather(x2v, [di]) - plsc.load_gather(x2v, [si])
        dsqv[pl.ds(o, 16)] = d0 * d0 + d1 * d1 + d2 * d2
        return carry

    lax.fori_loop(0, EPT // 16, body, 0)
    pltpu.sync_copy(dsqv, dsq_hbm.at[pl.ds(base, EPT)])


@functools.partial(
    pl.kernel,
    out_type=jax.ShapeDtypeStruct((E2, H), jnp.float32),
    mesh=_MESH,
    scratch_types=[
        pltpu.VMEM((IRPT, IW), jnp.int32),
        pltpu.VMEM((IRPT, IW), jnp.int32),
        pltpu.VMEM((CG, H), jnp.float32),
        pltpu.VMEM((CG, H), jnp.float32),
        pltpu.SemaphoreType.DMA,
    ],
)
def _gather_kernel(a_hbm, b_hbm, dst2_hbm, src2_hbm, z_hbm,
                   dstv, srcv, av, bv, sem):
    wid = lax.axis_index("s") * NC + lax.axis_index("c")
    pltpu.sync_copy(dst2_hbm.at[pl.ds(wid * IRPT, IRPT)], dstv)
    pltpu.sync_copy(src2_hbm.at[pl.ds(wid * IRPT, IRPT)], srcv)
    nsub = CG // IW

    def chunk(g, carry):
        base = wid * EPT + g * CG
        cps = []
        for k in range(nsub):
            cps.append(pltpu.async_copy(
                a_hbm.at[dstv.at[g * nsub + k]],
                av.at[pl.ds(k * IW, IW)], sem))
            cps.append(pltpu.async_copy(
                b_hbm.at[srcv.at[g * nsub + k]],
                bv.at[pl.ds(k * IW, IW)], sem))
        for cp in cps:
            cp.wait()

        def row(i, c2):
            for j in range(H // 16):
                sl = pl.ds(j * 16, 16)
                av[i, sl] = jnp.maximum(av[i, sl] + bv[i, sl], 0.0)
            return c2

        lax.fori_loop(0, CG, row, 0)
        pltpu.sync_copy(av, z_hbm.at[pl.ds(base, CG)])
        return carry

    lax.fori_loop(0, GG, chunk, 0)


def _edge_mlp_body(z_ref, dsq_ref, w2_ref, b2_ref, winf_ref, binf_ref, w_ref):
    z = z_ref[...]
    m = jnp.maximum(
        jnp.dot(z, w2_ref[...], preferred_element_type=jnp.float32) + b2_ref[...], 0.0)
    eij = jnp.dot(m, winf_ref[...], preferred_element_type=jnp.float32) + binf_ref[...]
    dsq = dsq_ref[...]
    edge_dis = jax.nn.sigmoid(30.0 / (jnp.sqrt(dsq) + 1e-08))
    ew = jax.nn.sigmoid(eij * edge_dis)
    w_ref[...] = m * ew


def _edge_mlp(z, dsq1, w2, b2, winf, binf):
    be = 2560
    return pl.pallas_call(
        _edge_mlp_body,
        grid=(E2 // be,),
        in_specs=[
            pl.BlockSpec((be, H), lambda i: (i, 0)),
            pl.BlockSpec((be, 1), lambda i: (i, 0)),
            pl.BlockSpec((H, H), lambda i: (0, 0)),
            pl.BlockSpec((1, H), lambda i: (0, 0)),
            pl.BlockSpec((H, 1), lambda i: (0, 0)),
            pl.BlockSpec((1, 1), lambda i: (0, 0)),
        ],
        out_specs=pl.BlockSpec((be, H), lambda i: (i, 0)),
        out_shape=jax.ShapeDtypeStruct((E2, H), jnp.float32),
    )(z, dsq1, w2, b2, winf, binf)


@functools.partial(
    pl.kernel,
    out_type=jax.ShapeDtypeStruct((NC, NPAD, H), jnp.float32),
    mesh=_MESH,
    scratch_types=[
        pltpu.VMEM((IRPT, IW), jnp.int32),
        pltpu.VMEM((CS, H), jnp.float32),
        pltpu.VMEM_SHARED((NPAD, H), jnp.float32),
    ],
)
def _scatter_kernel(w_hbm, dst2_hbm, out_hbm, dstv, wv, acc):
    c = lax.axis_index("c")
    s = lax.axis_index("s")
    wid = s * NC + c
    myrow = s * RPT
    nsub = CS // IW

    pltpu.sync_copy(dst2_hbm.at[pl.ds(wid * IRPT, IRPT)], dstv)

    def zrow(i, carry):
        for j in range(H // 16):
            wv[i, pl.ds(j * 16, 16)] = jnp.zeros((16,), jnp.float32)
        return carry

    lax.fori_loop(0, IW, zrow, 0)
    for t in range(RPT // IW):
        pltpu.sync_copy(wv.at[pl.ds(0, IW)],
                        acc.at[pl.ds(myrow + t * IW, IW)])
    plsc.subcore_barrier()

    def chunk(g, carry):
        base = wid * EPT + g * CS
        pltpu.sync_copy(w_hbm.at[pl.ds(base, CS)], wv)
        for k in range(nsub):
            pltpu.sync_copy(wv.at[pl.ds(k * IW, IW)],
                            acc.at[dstv.at[g * nsub + k]], add=True)
        return carry

    lax.fori_loop(0, GS, chunk, 0)
    plsc.subcore_barrier()

    for t in range(RPT // IW):
        pltpu.sync_copy(acc.at[pl.ds(myrow + t * IW, IW)],
                        wv.at[pl.ds(0, IW)])
        pltpu.sync_copy(wv.at[pl.ds(0, IW)],
                        out_hbm.at[c].at[pl.ds(myrow + t * IW, IW)])


def _node_mlp_body(p0_ref, p1_ref, h_ref, wt_ref, wb_ref, bn1_ref, w2_ref,
                   bn2_ref, g_ref, b_ref, out_ref):
    mi = p0_ref[...] + p1_ref[...]
    hb = h_ref[...]
    t = jnp.maximum(
        jnp.dot(mi, wt_ref[...], preferred_element_type=jnp.float32)
        + jnp.dot(hb, wb_ref[...], preferred_element_type=jnp.float32)
        + bn1_ref[...], 0.0)
    upd = jnp.dot(t, w2_ref[...], preferred_element_type=jnp.float32) + bn2_ref[...]
    hn = hb + upd
    mu = jnp.mean(hn, axis=-1, keepdims=True)
    var = jnp.mean((hn - mu) ** 2, axis=-1, keepdims=True)
    out_ref[...] = (hn - mu) * lax.rsqrt(var + 1e-05) * g_ref[...] + b_ref[...]


def _node_mlp(p0, p1, hp, wt, wb, bn1, w2, bn2, g, b):
    bn = 512
    return pl.pallas_call(
        _node_mlp_body,
        grid=(NPAD // bn,),
        in_specs=[
            pl.BlockSpec((bn, H), lambda i: (i, 0)),
            pl.BlockSpec((bn, H), lambda i: (i, 0)),
            pl.BlockSpec((bn, H), lambda i: (i, 0)),
            pl.BlockSpec((H, H), lambda i: (0, 0)),
            pl.BlockSpec((H, H), lambda i: (0, 0)),
            pl.BlockSpec((1, H), lambda i: (0, 0)),
            pl.BlockSpec((H, H), lambda i: (0, 0)),
            pl.BlockSpec((1, H), lambda i: (0, 0)),
            pl.BlockSpec((1, H), lambda i: (0, 0)),
            pl.BlockSpec((1, H), lambda i: (0, 0)),
        ],
        out_specs=pl.BlockSpec((bn, H), lambda i: (i, 0)),
        out_shape=jax.ShapeDtypeStruct((NPAD, H), jnp.float32),
    )(p0, p1, hp, wt, wb, bn1, w2, bn2, g, b)


def kernel(h, x, edge_index, We1, be1, We2, be2, Winf, binf, Wn1, bn1, Wn2,
           bn2, ln_g, ln_b):
    pad_id = jnp.full((E2 - E,), NPAD - 1, dtype=jnp.int32)
    src = jnp.concatenate([edge_index[0], pad_id])
    dst = jnp.concatenate([edge_index[1], pad_id])
    dst2 = dst.reshape(E2 // IW, IW)
    src2 = src.reshape(E2 // IW, IW)

    hp = jnp.pad(h, ((0, NPAD - N), (0, 0)))
    a, b = _pre_ab(hp, We1[:H], We1[H:], be1.reshape(1, H))

    xp = jnp.pad(x, ((0, NPAD - N), (0, 0)))
    dsq = _dsq_kernel(xp[:, 0], xp[:, 1], xp[:, 2], dst, src)

    z = _gather_kernel(a, b, dst2, src2)

    w = _edge_mlp(z, dsq.reshape(E2, 1), We2, be2.reshape(1, H), Winf,
                  binf.reshape(1, 1))

    parts = _scatter_kernel(w, dst2)

    hn = _node_mlp(parts[0], parts[1], hp, Wn1[:H], Wn1[H:],
                   bn1.reshape(1, H), Wn2, bn2.reshape(1, H),
                   ln_g.reshape(1, H), ln_b.reshape(1, H))
    return (hn[:N], x)


# speedup vs baseline: 3.8090x; 1.0309x over previous
import functools

import jax
import jax.numpy as jnp
from jax import lax
from jax.experimental import pallas as pl
from jax.experimental.pallas import tpu as pltpu
from jax.experimental.pallas import tpu_sc as plsc

N = 10000
H = 128
E = 320000
NPAD = 10240
NC = 2
NS = 16
NW = NC * NS
EPT = 10240
E2 = NW * EPT

IW = 128
IRPT = EPT // IW
CG = 256
GG = EPT // CG
CS = 256
GS = EPT // CS
RPT = NPAD // NS

_MESH = plsc.VectorSubcoreMesh(
    core_axis_name="c", subcore_axis_name="s", num_cores=NC, num_subcores=NS)


def _pre_ab_body(h_ref, wt_ref, wb_ref, be1_ref, a_ref, b_ref):
    hb = h_ref[...]
    a_ref[...] = jnp.dot(hb, wt_ref[...], preferred_element_type=jnp.float32) + be1_ref[...]
    b_ref[...] = jnp.dot(hb, wb_ref[...], preferred_element_type=jnp.float32)


def _pre_ab(hp, wt, wb, be1):
    bn = 512
    return pl.pallas_call(
        _pre_ab_body,
        grid=(NPAD // bn,),
        in_specs=[
            pl.BlockSpec((bn, H), lambda i: (i, 0)),
            pl.BlockSpec((H, H), lambda i: (0, 0)),
            pl.BlockSpec((H, H), lambda i: (0, 0)),
            pl.BlockSpec((1, H), lambda i: (0, 0)),
        ],
        out_specs=[
            pl.BlockSpec((bn, H), lambda i: (i, 0)),
            pl.BlockSpec((bn, H), lambda i: (i, 0)),
        ],
        out_shape=[
            jax.ShapeDtypeStruct((NPAD, H), jnp.float32),
            jax.ShapeDtypeStruct((NPAD, H), jnp.float32),
        ],
    )(hp, wt, wb, be1)


@functools.partial(
    pl.kernel,
    out_type=jax.ShapeDtypeStruct((E2,), jnp.float32),
    mesh=_MESH,
    scratch_types=[
        pltpu.VMEM((NPAD,), jnp.float32),
        pltpu.VMEM((NPAD,), jnp.float32),
        pltpu.VMEM((NPAD,), jnp.float32),
        pltpu.VMEM((EPT,), jnp.int32),
        pltpu.VMEM((EPT,), jnp.int32),
        pltpu.VMEM((EPT,), jnp.float32),
    ],
    compiler_params=pltpu.CompilerParams(needs_layout_passes=False),
)
def _dsq_kernel(x0_hbm, x1_hbm, x2_hbm, dst_hbm, src_hbm, dsq_hbm,
                x0v, x1v, x2v, dstv, srcv, dsqv):
    wid = lax.axis_index("s") * NC + lax.axis_index("c")
    base = wid * EPT
    pltpu.sync_copy(x0_hbm, x0v)
    pltpu.sync_copy(x1_hbm, x1v)
    pltpu.sync_copy(x2_hbm, x2v)
    pltpu.sync_copy(dst_hbm.at[pl.ds(base, EPT)], dstv)
    pltpu.sync_copy(src_hbm.at[pl.ds(base, EPT)], srcv)

    def body(t, carry):
        o = t * 16
        di = dstv[pl.ds(o, 16)]
        si = srcv[pl.ds(o, 16)]
        d0 = plsc.load_gather(x0v, [di]) - plsc.load_gather(x0v, [si])
        d1 = plsc.load_gather(x1v, [di]) - plsc.load_gather(x1v, [si])
        d2 = plsc.load_gather(x2v, [di]) - plsc.load_gather(x2v, [si])
        dsqv[pl.ds(o, 16)] = d0 * d0 + d1 * d1 + d2 * d2
        return carry

    lax.fori_loop(0, EPT // 16, body, 0)
    pltpu.sync_copy(dsqv, dsq_hbm.at[pl.ds(base, EPT)])


GP = EPT // IW


@functools.partial(
    pl.kernel,
    out_type=jax.ShapeDtypeStruct((E2, H), jnp.float32),
    mesh=_MESH,
    scratch_types=[
        pltpu.VMEM((IRPT, IW), jnp.int32),
        pltpu.VMEM((IRPT, IW), jnp.int32),
        pltpu.VMEM((IW, H), jnp.float32),
        pltpu.VMEM((IW, H), jnp.float32),
        pltpu.VMEM((IW, H), jnp.float32),
        pltpu.VMEM((IW, H), jnp.float32),
        pltpu.SemaphoreType.DMA,
        pltpu.SemaphoreType.DMA,
        pltpu.SemaphoreType.DMA,
        pltpu.SemaphoreType.DMA,
    ],
)
def _gather_kernel(a_hbm, b_hbm, dst2_hbm, src2_hbm, z_hbm,
                   dstv, srcv, av0, av1, bv0, bv1, gs0, gs1, ss0, ss1):
    wid = lax.axis_index("s") * NC + lax.axis_index("c")
    pltpu.sync_copy(dst2_hbm.at[pl.ds(wid * IRPT, IRPT)], dstv)
    pltpu.sync_copy(src2_hbm.at[pl.ds(wid * IRPT, IRPT)], srcv)
    avs = (av0, av1)
    bvs = (bv0, bv1)
    gss = (gs0, gs1)
    sss = (ss0, ss1)

    def issue_gather(g, s):
        pltpu.async_copy(a_hbm.at[dstv.at[g]], avs[s], gss[s])
        pltpu.async_copy(b_hbm.at[srcv.at[g]], bvs[s], gss[s])

    def wait_gather(s):
        pltpu.make_async_copy(a_hbm.at[dstv.at[0]], avs[s], gss[s]).wait()
        pltpu.make_async_copy(b_hbm.at[srcv.at[0]], bvs[s], gss[s]).wait()

    def compute(s):
        av, bv = avs[s], bvs[s]

        @plsc.parallel_loop(0, IW, 1, unroll=4)
        def _row(i):
            for j in range(H // 16):
                sl = pl.ds(j * 16, 16)
                av[i, sl] = jnp.maximum(av[i, sl] + bv[i, sl], 0.0)

    def issue_store(g, s):
        pltpu.async_copy(avs[s], z_hbm.at[pl.ds(wid * EPT + g * IW, IW)],
                         sss[s])

    def wait_store(s):
        pltpu.make_async_copy(avs[s], z_hbm.at[pl.ds(0, IW)], sss[s]).wait()

    issue_gather(0, 0)
    wait_gather(0)
    issue_gather(1, 1)
    compute(0)
    issue_store(0, 0)

    def pair(k, carry):
        g1 = 1 + 2 * k
        wait_gather(1)
        wait_store(0)
        issue_gather(g1 + 1, 0)
        compute(1)
        issue_store(g1, 1)
        g2 = g1 + 1
        wait_gather(0)
        wait_store(1)
        issue_gather(g2 + 1, 1)
        compute(0)
        issue_store(g2, 0)
        return carry

    lax.fori_loop(0, (GP - 2) // 2, pair, 0)

    wait_gather(1)
    compute(1)
    issue_store(GP - 1, 1)
    wait_store(0)
    wait_store(1)


def _edge_mlp_body(z_ref, dsq_ref, w2_ref, b2_ref, winf_ref, binf_ref, w_ref):
    z = z_ref[...]
    m = jnp.maximum(
        jnp.dot(z, w2_ref[...], preferred_element_type=jnp.float32) + b2_ref[...], 0.0)
    eij = jnp.dot(m, winf_ref[...], preferred_element_type=jnp.float32) + binf_ref[...]
    dsq = dsq_ref[...]
    edge_dis = jax.nn.sigmoid(30.0 / (jnp.sqrt(dsq) + 1e-08))
    ew = jax.nn.sigmoid(eij * edge_dis)
    w_ref[...] = m * ew


def _edge_mlp(z, dsq1, w2, b2, winf, binf):
    be = 2560
    return pl.pallas_call(
        _edge_mlp_body,
        grid=(E2 // be,),
        in_specs=[
            pl.BlockSpec((be, H), lambda i: (i, 0)),
            pl.BlockSpec((be, 1), lambda i: (i, 0)),
            pl.BlockSpec((H, H), lambda i: (0, 0)),
            pl.BlockSpec((1, H), lambda i: (0, 0)),
            pl.BlockSpec((H, 1), lambda i: (0, 0)),
            pl.BlockSpec((1, 1), lambda i: (0, 0)),
        ],
        out_specs=pl.BlockSpec((be, H), lambda i: (i, 0)),
        out_shape=jax.ShapeDtypeStruct((E2, H), jnp.float32),
    )(z, dsq1, w2, b2, winf, binf)


@functools.partial(
    pl.kernel,
    out_type=jax.ShapeDtypeStruct((NC, NPAD, H), jnp.float32),
    mesh=_MESH,
    scratch_types=[
        pltpu.VMEM((IRPT, IW), jnp.int32),
        pltpu.VMEM((CS, H), jnp.float32),
        pltpu.VMEM_SHARED((NPAD, H), jnp.float32),
    ],
)
def _scatter_kernel(w_hbm, dst2_hbm, out_hbm, dstv, wv, acc):
    c = lax.axis_index("c")
    s = lax.axis_index("s")
    wid = s * NC + c
    myrow = s * RPT
    nsub = CS // IW

    pltpu.sync_copy(dst2_hbm.at[pl.ds(wid * IRPT, IRPT)], dstv)

    def zrow(i, carry):
        for j in range(H // 16):
            wv[i, pl.ds(j * 16, 16)] = jnp.zeros((16,), jnp.float32)
        return carry

    lax.fori_loop(0, IW, zrow, 0)
    for t in range(RPT // IW):
        pltpu.sync_copy(wv.at[pl.ds(0, IW)],
                        acc.at[pl.ds(myrow + t * IW, IW)])
    plsc.subcore_barrier()

    def chunk(g, carry):
        base = wid * EPT + g * CS
        pltpu.sync_copy(w_hbm.at[pl.ds(base, CS)], wv)
        for k in range(nsub):
            pltpu.sync_copy(wv.at[pl.ds(k * IW, IW)],
                            acc.at[dstv.at[g * nsub + k]], add=True)
        return carry

    lax.fori_loop(0, GS, chunk, 0)
    plsc.subcore_barrier()

    for t in range(RPT // IW):
        pltpu.sync_copy(acc.at[pl.ds(myrow + t * IW, IW)],
                        wv.at[pl.ds(0, IW)])
        pltpu.sync_copy(wv.at[pl.ds(0, IW)],
                        out_hbm.at[c].at[pl.ds(myrow + t * IW, IW)])


def _node_mlp_body(p0_ref, p1_ref, h_ref, wt_ref, wb_ref, bn1_ref, w2_ref,
                   bn2_ref, g_ref, b_ref, out_ref):
    mi = p0_ref[...] + p1_ref[...]
    hb = h_ref[...]
    t = jnp.maximum(
        jnp.dot(mi, wt_ref[...], preferred_element_type=jnp.float32)
        + jnp.dot(hb, wb_ref[...], preferred_element_type=jnp.float32)
        + bn1_ref[...], 0.0)
    upd = jnp.dot(t, w2_ref[...], preferred_element_type=jnp.float32) + bn2_ref[...]
    hn = hb + upd
    mu = jnp.mean(hn, axis=-1, keepdims=True)
    var = jnp.mean((hn - mu) ** 2, axis=-1, keepdims=True)
    out_ref[...] = (hn - mu) * lax.rsqrt(var + 1e-05) * g_ref[...] + b_ref[...]


def _node_mlp(p0, p1, hp, wt, wb, bn1, w2, bn2, g, b):
    bn = 512
    return pl.pallas_call(
        _node_mlp_body,
        grid=(NPAD // bn,),
        in_specs=[
            pl.BlockSpec((bn, H), lambda i: (i, 0)),
            pl.BlockSpec((bn, H), lambda i: (i, 0)),
            pl.BlockSpec((bn, H), lambda i: (i, 0)),
            pl.BlockSpec((H, H), lambda i: (0, 0)),
            pl.BlockSpec((H, H), lambda i: (0, 0)),
            pl.BlockSpec((1, H), lambda i: (0, 0)),
            pl.BlockSpec((H, H), lambda i: (0, 0)),
            pl.BlockSpec((1, H), lambda i: (0, 0)),
            pl.BlockSpec((1, H), lambda i: (0, 0)),
            pl.BlockSpec((1, H), lambda i: (0, 0)),
        ],
        out_specs=pl.BlockSpec((bn, H), lambda i: (i, 0)),
        out_shape=jax.ShapeDtypeStruct((NPAD, H), jnp.float32),
    )(p0, p1, hp, wt, wb, bn1, w2, bn2, g, b)


def kernel(h, x, edge_index, We1, be1, We2, be2, Winf, binf, Wn1, bn1, Wn2,
           bn2, ln_g, ln_b):
    pad_id = jnp.full((E2 - E,), NPAD - 1, dtype=jnp.int32)
    src = jnp.concatenate([edge_index[0], pad_id])
    dst = jnp.concatenate([edge_index[1], pad_id])
    dst2 = dst.reshape(E2 // IW, IW)
    src2 = src.reshape(E2 // IW, IW)

    hp = jnp.pad(h, ((0, NPAD - N), (0, 0)))
    a, b = _pre_ab(hp, We1[:H], We1[H:], be1.reshape(1, H))

    xp = jnp.pad(x, ((0, NPAD - N), (0, 0)))
    dsq = _dsq_kernel(xp[:, 0], xp[:, 1], xp[:, 2], dst, src)

    z = _gather_kernel(a, b, dst2, src2)

    w = _edge_mlp(z, dsq.reshape(E2, 1), We2, be2.reshape(1, H), Winf,
                  binf.reshape(1, 1))

    parts = _scatter_kernel(w, dst2)

    hn = _node_mlp(parts[0], parts[1], hp, Wn1[:H], Wn1[H:],
                   bn1.reshape(1, H), Wn2, bn2.reshape(1, H),
                   ln_g.reshape(1, H), ln_b.reshape(1, H))
    return (hn[:N], x)
